# baseline (speedup 1.0000x reference)
import functools

import jax
import jax.numpy as jnp
from jax import lax
from jax.experimental import pallas as pl
from jax.experimental.pallas import tpu as pltpu
from jax.experimental.pallas import tpu_sc as plsc

_B, _L, _V, _D = 4096, 200, 32, 129
_NW = 32
_BC = 256
_NBT = _B // _BC


def _build_table(token_table, ooba_table, pos_table):
    def body(tok_ref, ooba_ref, pos_ref, tbl_ref):
        comb = jnp.concatenate([tok_ref[...], ooba_ref[...]], axis=-1)
        tbl_ref[...] = comb.T[None, :, :] + pos_ref[...][:, :, None]

    return pl.pallas_call(
        body,
        out_shape=jax.ShapeDtypeStruct((_L, _D, _V), jnp.float32),
    )(token_table, ooba_table, pos_table)


def _sc_lookup(tbl, xt):
    mesh = plsc.VectorSubcoreMesh(core_axis_name="c", subcore_axis_name="s")

    @functools.partial(
        pl.kernel,
        mesh=mesh,
        out_type=jax.ShapeDtypeStruct((_D, _L, _B), jnp.float32),
        scratch_types=[
            pltpu.VMEM((_D * _V,), jnp.float32),
            pltpu.VMEM((_B,), jnp.int32),
            [pltpu.VMEM((_D, 1, _BC), jnp.float32)] * 2,
            [pltpu.SemaphoreType.DMA] * 2,
        ],
        compiler_params=pltpu.CompilerParams(needs_layout_passes=False),
    )
    def k(tbl_hbm, xt_hbm, out_hbm, tbl_v, xv, buf, sem):
        wid = lax.axis_index("s") * 2 + lax.axis_index("c")
        l_lo = wid * 6 + jnp.minimum(wid, 8)
        l_hi = l_lo + 6 + jnp.where(wid < 8, 1, 0)

        def fill(bt, p):
            xs = [xv[pl.ds(bt * _BC + j * 16, 16)] for j in range(_BC // 16)]

            @plsc.parallel_loop(0, _D, unroll=4)
            def dbody(d):
                for j in range(_BC // 16):
                    vals = plsc.load_gather(tbl_v, [xs[j] + d * _V])
                    buf[p][d, 0, pl.ds(j * 16, 16)] = vals

        def flush_start(l, bt, p):
            pltpu.async_copy(
                buf[p],
                out_hbm.at[pl.ds(0, _D), pl.ds(l, 1), pl.ds(bt * _BC, _BC)],
                sem[p])

        def flush_wait(l, bt, p):
            pltpu.make_async_copy(
                buf[p],
                out_hbm.at[pl.ds(0, _D), pl.ds(l, 1), pl.ds(bt * _BC, _BC)],
                sem[p]).wait()

        def per_l(l, carry):
            pltpu.sync_copy(tbl_hbm.at[pl.ds(l * _D * _V, _D * _V)], tbl_v)
            pltpu.sync_copy(xt_hbm.at[pl.ds(l * _B, _B)], xv)
            fill(0, 0)
            flush_start(l, 0, 0)
            fill(1, 1)
            flush_start(l, 1, 1)

            def pair(h, c):
                bt = 2 * h
                flush_wait(l, bt, 0)
                fill(bt + 2, 0)
                flush_start(l, bt + 2, 0)
                flush_wait(l, bt + 1, 1)
                fill(bt + 3, 1)
                flush_start(l, bt + 3, 1)
                return c

            lax.fori_loop(0, _NBT // 2 - 1, pair, 0)
            flush_wait(l, _NBT - 2, 0)
            flush_wait(l, _NBT - 1, 1)
            return carry

        lax.fori_loop(l_lo, l_hi, per_l, 0)

    return k(tbl, xt)


def kernel(x, token_table, ooba_table, pos_table):
    tbl = _build_table(token_table, ooba_table, pos_table).reshape(-1)
    xt = x.T.reshape(-1).astype(jnp.int32)
    out = _sc_lookup(tbl, xt)
    return out.transpose(2, 1, 0)


# speedup vs baseline: 1.1084x; 1.1084x over previous
import functools

import jax
import jax.numpy as jnp
from jax import lax
from jax.experimental import pallas as pl
from jax.experimental.pallas import tpu as pltpu
from jax.experimental.pallas import tpu_sc as plsc

_B, _L, _V, _D = 4096, 200, 32, 129
_NW = 32
_BC = 128
_NBT = _B // _BC


def _build_table(token_table, ooba_table, pos_table):
    def body(tok_ref, ooba_ref, pos_ref, tbl_ref):
        comb = jnp.concatenate([tok_ref[...], ooba_ref[...]], axis=-1)
        tbl_ref[...] = comb.T[None, :, :] + pos_ref[...][:, :, None]

    return pl.pallas_call(
        body,
        out_shape=jax.ShapeDtypeStruct((_L, _D, _V), jnp.float32),
    )(token_table, ooba_table, pos_table)


def _sc_lookup(tbl, xt):
    mesh = plsc.VectorSubcoreMesh(core_axis_name="c", subcore_axis_name="s")

    @functools.partial(
        pl.kernel,
        mesh=mesh,
        out_type=jax.ShapeDtypeStruct((_D, _L, _B), jnp.float32),
        scratch_types=[
            pltpu.VMEM((_D * _V,), jnp.float32),
            pltpu.VMEM((_B,), jnp.int32),
            [pltpu.VMEM((_D, 1, _BC), jnp.float32)] * 2,
            [pltpu.SemaphoreType.DMA] * 2,
        ],
        compiler_params=pltpu.CompilerParams(needs_layout_passes=False),
    )
    def k(tbl_hbm, xt_hbm, out_hbm, tbl_v, xv, buf, sem):
        wid = lax.axis_index("s") * 2 + lax.axis_index("c")
        l_lo = wid * 6 + jnp.minimum(wid, 8)
        l_hi = l_lo + 6 + jnp.where(wid < 8, 1, 0)

        def fill(bt, p):
            xs = [xv[pl.ds(bt * _BC + j * 16, 16)] for j in range(_BC // 16)]

            @plsc.parallel_loop(0, _D, unroll=4)
            def dbody(d):
                for j in range(_BC // 16):
                    vals = plsc.load_gather(tbl_v, [xs[j] + d * _V])
                    buf[p][d, 0, pl.ds(j * 16, 16)] = vals

        def flush_start(l, bt, p):
            pltpu.async_copy(
                buf[p],
                out_hbm.at[pl.ds(0, _D), pl.ds(l, 1), pl.ds(bt * _BC, _BC)],
                sem[p])

        def flush_wait(l, bt, p):
            pltpu.make_async_copy(
                buf[p],
                out_hbm.at[pl.ds(0, _D), pl.ds(l, 1), pl.ds(bt * _BC, _BC)],
                sem[p]).wait()

        def per_l(l, carry):
            pltpu.sync_copy(tbl_hbm.at[pl.ds(l * _D * _V, _D * _V)], tbl_v)
            pltpu.sync_copy(xt_hbm.at[pl.ds(l * _B, _B)], xv)
            fill(0, 0)
            flush_start(l, 0, 0)
            fill(1, 1)
            flush_start(l, 1, 1)

            def pair(h, c):
                bt = 2 * h
                flush_wait(l, bt, 0)
                fill(bt + 2, 0)
                flush_start(l, bt + 2, 0)
                flush_wait(l, bt + 1, 1)
                fill(bt + 3, 1)
                flush_start(l, bt + 3, 1)
                return c

            lax.fori_loop(0, _NBT // 2 - 1, pair, 0)
            flush_wait(l, _NBT - 2, 0)
            flush_wait(l, _NBT - 1, 1)
            return carry

        lax.fori_loop(l_lo, l_hi, per_l, 0)

    return k(tbl, xt)


def kernel(x, token_table, ooba_table, pos_table):
    tbl = _build_table(token_table, ooba_table, pos_table).reshape(-1)
    xt = x.T.reshape(-1).astype(jnp.int32)
    out = _sc_lookup(tbl, xt)
    return out.transpose(2, 1, 0)


# speedup vs baseline: 1.1230x; 1.0132x over previous
import functools

import jax
import jax.numpy as jnp
from jax import lax
from jax.experimental import pallas as pl
from jax.experimental.pallas import tpu as pltpu
from jax.experimental.pallas import tpu_sc as plsc

_B, _L, _V, _D = 4096, 200, 32, 129
_NW = 32
_BC = 128
_NBT = _B // _BC


def _build_table(token_table, ooba_table, pos_table):
    def body(tok_ref, ooba_ref, pos_ref, tbl_ref):
        comb = jnp.concatenate([tok_ref[...], ooba_ref[...]], axis=-1)
        tbl_ref[...] = comb.T[None, :, :] + pos_ref[...][:, :, None]

    return pl.pallas_call(
        body,
        out_shape=jax.ShapeDtypeStruct((_L, _D, _V), jnp.float32),
    )(token_table, ooba_table, pos_table)


def _sc_lookup(tbl, xt):
    mesh = plsc.VectorSubcoreMesh(core_axis_name="c", subcore_axis_name="s")

    @functools.partial(
        pl.kernel,
        mesh=mesh,
        out_type=jax.ShapeDtypeStruct((_D, _L, _B), jnp.float32),
        scratch_types=[
            pltpu.VMEM((_D * _V,), jnp.float32),
            pltpu.VMEM((_B,), jnp.int32),
            [pltpu.VMEM((_D, 1, _BC), jnp.float32)] * 2,
            [pltpu.SemaphoreType.DMA] * 2,
        ],
        compiler_params=pltpu.CompilerParams(needs_layout_passes=False),
    )
    def k(tbl_hbm, xt_hbm, out_hbm, tbl_v, xv, buf, sem):
        wid = lax.axis_index("s") * 2 + lax.axis_index("c")
        l_lo = wid * 6 + jnp.minimum(wid, 8)
        l_hi = l_lo + 6 + jnp.where(wid < 8, 1, 0)

        def fill(bt, p):
            xs = [xv[pl.ds(bt * _BC + j * 16, 16)] for j in range(_BC // 16)]

            @plsc.parallel_loop(0, _D, unroll=8)
            def dbody(d):
                for j in range(_BC // 16):
                    vals = plsc.load_gather(tbl_v, [xs[j] + d * _V])
                    buf[p][d, 0, pl.ds(j * 16, 16)] = vals

        def flush_start(l, bt, p):
            pltpu.async_copy(
                buf[p],
                out_hbm.at[pl.ds(0, _D), pl.ds(l, 1), pl.ds(bt * _BC, _BC)],
                sem[p])

        def flush_wait(l, bt, p):
            pltpu.make_async_copy(
                buf[p],
                out_hbm.at[pl.ds(0, _D), pl.ds(l, 1), pl.ds(bt * _BC, _BC)],
                sem[p]).wait()

        def per_l(l, carry):
            pltpu.sync_copy(tbl_hbm.at[pl.ds(l * _D * _V, _D * _V)], tbl_v)
            pltpu.sync_copy(xt_hbm.at[pl.ds(l * _B, _B)], xv)
            fill(0, 0)
            flush_start(l, 0, 0)
            fill(1, 1)
            flush_start(l, 1, 1)

            def pair(h, c):
                bt = 2 * h
                flush_wait(l, bt, 0)
                fill(bt + 2, 0)
                flush_start(l, bt + 2, 0)
                flush_wait(l, bt + 1, 1)
                fill(bt + 3, 1)
                flush_start(l, bt + 3, 1)
                return c

            lax.fori_loop(0, _NBT // 2 - 1, pair, 0)
            flush_wait(l, _NBT - 2, 0)
            flush_wait(l, _NBT - 1, 1)
            return carry

        lax.fori_loop(l_lo, l_hi, per_l, 0)

    return k(tbl, xt)


def kernel(x, token_table, ooba_table, pos_table):
    tbl = _build_table(token_table, ooba_table, pos_table).reshape(-1)
    xt = x.T.reshape(-1).astype(jnp.int32)
    out = _sc_lookup(tbl, xt)
    return out.transpose(2, 1, 0)
